# baseline (speedup 1.0000x reference)
import functools

import jax
import jax.numpy as jnp
from jax import lax
from jax.experimental import pallas as pl
from jax.experimental.pallas import tpu as pltpu
from jax.experimental.pallas import tpu_sc as plsc

_N = 8192
_NPL = 1024
_NUM_LEVELS = 8
_E = _N - _NPL
_B = 16
_L = _N + _E
_D = 128
_RPL = _NPL * _B
_NROWS = _L * _B
_BLK = 2048
_LBLKS = _RPL // _BLK
_EBOFF = _N * _B // _BLK


def _lv0_kernel(a_ref, w_ref, big_ref):
    big_ref[...] = jnp.dot(a_ref[...], w_ref[...],
                           preferred_element_type=jnp.float32)


def _level_kernel(g_ref, dst_ref, edge_ref, ws_ref, wdc_ref, wec_ref,
                  big_in_ref, big_ref):
    del big_in_ref
    big_ref[...] = (
        jnp.dot(g_ref[...], ws_ref[...], preferred_element_type=jnp.float32)
        + jnp.dot(dst_ref[...], wdc_ref[...],
                  preferred_element_type=jnp.float32)
        + jnp.dot(edge_ref[...], wec_ref[...],
                  preferred_element_type=jnp.float32))


@functools.lru_cache(maxsize=None)
def _make_sc_gather():
    info = plsc.get_sparse_core_info()
    nc, ns = info.num_cores, info.num_subcores
    bpw = _RPL // (nc * ns)
    mesh = plsc.VectorSubcoreMesh(core_axis_name="c", subcore_axis_name="s")

    @functools.partial(
        pl.kernel, mesh=mesh,
        out_type=jax.ShapeDtypeStruct((_RPL, _D), jnp.float32),
        scratch_types=[
            pltpu.VMEM((bpw,), jnp.int32),
            pltpu.VMEM((bpw, _D), jnp.float32),
            pltpu.SemaphoreType.DMA,
        ],
    )
    def sc_gather(table_hbm, idx_hbm, out_hbm, idx_v, rows_v, sem):
        wid = lax.axis_index("s") * nc + lax.axis_index("c")
        base = wid * bpw
        pltpu.sync_copy(idx_hbm.at[pl.ds(base, bpw)], idx_v)
        pltpu.async_copy(table_hbm.at[idx_v], rows_v, sem).wait()
        pltpu.sync_copy(rows_v, out_hbm.at[pl.ds(base, bpw)])

    return sc_gather


def _gather(table, idx):
    return _make_sc_gather()(table, idx)


def _off(o):
    return lambda i: (o + i, 0)


_W128 = pl.BlockSpec((_D, _D), lambda i: (0, 0))


def kernel(connectivitys, all_seg_ids, all_feats, W_lin, W_msg):
    del all_seg_ids
    a = all_feats.reshape(_NROWS, _D)
    wlin_t = W_lin.T
    ws_t = W_msg[:, :_D].T
    wdc_t = wlin_t @ W_msg[:, _D:2 * _D].T
    wec_t = wlin_t @ W_msg[:, 2 * _D:].T

    src = connectivitys[:, 0, :].astype(jnp.int32)
    idx_all = (src.T * _B
               + jnp.arange(_B, dtype=jnp.int32)[None, :]).reshape(_E * _B)

    def _ae_map(i):
        return (jnp.where(i < _LBLKS, i, _EBOFF - _LBLKS + i), 0)

    big = pl.pallas_call(
        _lv0_kernel,
        grid=(_LBLKS + _E * _B // _BLK,),
        in_specs=[pl.BlockSpec((_BLK, _D), _ae_map), _W128],
        out_specs=pl.BlockSpec((_BLK, _D), _ae_map),
        out_shape=jax.ShapeDtypeStruct((_NROWS, _D), jnp.float32),
    )(a, wlin_t)

    for l in range(1, _NUM_LEVELS):
        g = _gather(big, idx_all[(l - 1) * _RPL:l * _RPL])
        big = pl.pallas_call(
            _level_kernel,
            grid=(_LBLKS,),
            in_specs=[pl.BlockSpec((_BLK, _D), _off(0)),
                      pl.BlockSpec((_BLK, _D), _off(l * _LBLKS)),
                      pl.BlockSpec((_BLK, _D),
                                   _off(_EBOFF + (l - 1) * _LBLKS)),
                      _W128, _W128, _W128,
                      pl.BlockSpec(memory_space=pl.ANY)],
            out_specs=pl.BlockSpec((_BLK, _D), _off(l * _LBLKS)),
            out_shape=jax.ShapeDtypeStruct((_NROWS, _D), jnp.float32),
            input_output_aliases={6: 0},
        )(g, a, a, ws_t, wdc_t, wec_t, big)

    return big.reshape(_L, _B, _D)


# speedup vs baseline: 1.2022x; 1.2022x over previous
import functools

import jax
import jax.numpy as jnp
from jax import lax
from jax.experimental import pallas as pl
from jax.experimental.pallas import tpu as pltpu
from jax.experimental.pallas import tpu_sc as plsc

_N = 8192
_NPL = 1024
_NUM_LEVELS = 8
_E = _N - _NPL
_B = 16
_L = _N + _E
_D = 128
_RPL = _NPL * _B
_NROWS = _L * _B
_BLK = 8192
_LBLKS = _RPL // _BLK
_EBOFF = _N * _B // _BLK


def _lv0_kernel(a_ref, w_ref, big_ref):
    big_ref[...] = jnp.dot(a_ref[...], w_ref[...],
                           preferred_element_type=jnp.float32)


def _level_kernel(g_ref, dst_ref, edge_ref, ws_ref, wdc_ref, wec_ref,
                  big_in_ref, big_ref):
    del big_in_ref
    big_ref[...] = (
        jnp.dot(g_ref[...], ws_ref[...], preferred_element_type=jnp.float32)
        + jnp.dot(dst_ref[...], wdc_ref[...],
                  preferred_element_type=jnp.float32)
        + jnp.dot(edge_ref[...], wec_ref[...],
                  preferred_element_type=jnp.float32))


@functools.lru_cache(maxsize=None)
def _make_sc_gather():
    info = plsc.get_sparse_core_info()
    nc, ns = info.num_cores, info.num_subcores
    bpw = _RPL // (nc * ns)
    mesh = plsc.VectorSubcoreMesh(core_axis_name="c", subcore_axis_name="s")

    @functools.partial(
        pl.kernel, mesh=mesh,
        out_type=jax.ShapeDtypeStruct((_RPL, _D), jnp.float32),
        scratch_types=[
            pltpu.VMEM((bpw,), jnp.int32),
            pltpu.VMEM((bpw, _D), jnp.float32),
            pltpu.SemaphoreType.DMA,
        ],
    )
    def sc_gather(table_hbm, idx_hbm, out_hbm, idx_v, rows_v, sem):
        wid = lax.axis_index("s") * nc + lax.axis_index("c")
        base = wid * bpw
        pltpu.sync_copy(idx_hbm.at[pl.ds(base, bpw)], idx_v)
        pltpu.async_copy(table_hbm.at[idx_v], rows_v, sem).wait()
        pltpu.sync_copy(rows_v, out_hbm.at[pl.ds(base, bpw)])

    return sc_gather


def _gather(table, idx):
    return _make_sc_gather()(table, idx)


def _off(o):
    return lambda i: (o + i, 0)


_W128 = pl.BlockSpec((_D, _D), lambda i: (0, 0))


def kernel(connectivitys, all_seg_ids, all_feats, W_lin, W_msg):
    del all_seg_ids
    a = all_feats.reshape(_NROWS, _D)
    wlin_t = W_lin.T
    ws_t = W_msg[:, :_D].T
    wdc_t = wlin_t @ W_msg[:, _D:2 * _D].T
    wec_t = wlin_t @ W_msg[:, 2 * _D:].T

    src = connectivitys[:, 0, :].astype(jnp.int32)
    idx_all = (src.T * _B
               + jnp.arange(_B, dtype=jnp.int32)[None, :]).reshape(_E * _B)

    def _ae_map(i):
        return (jnp.where(i < _LBLKS, i, _EBOFF - _LBLKS + i), 0)

    big = pl.pallas_call(
        _lv0_kernel,
        grid=(_LBLKS + _E * _B // _BLK,),
        in_specs=[pl.BlockSpec((_BLK, _D), _ae_map), _W128],
        out_specs=pl.BlockSpec((_BLK, _D), _ae_map),
        out_shape=jax.ShapeDtypeStruct((_NROWS, _D), jnp.float32),
    )(a, wlin_t)

    for l in range(1, _NUM_LEVELS):
        g = _gather(big, idx_all[(l - 1) * _RPL:l * _RPL])
        big = pl.pallas_call(
            _level_kernel,
            grid=(_LBLKS,),
            in_specs=[pl.BlockSpec((_BLK, _D), _off(0)),
                      pl.BlockSpec((_BLK, _D), _off(l * _LBLKS)),
                      pl.BlockSpec((_BLK, _D),
                                   _off(_EBOFF + (l - 1) * _LBLKS)),
                      _W128, _W128, _W128,
                      pl.BlockSpec(memory_space=pl.ANY)],
            out_specs=pl.BlockSpec((_BLK, _D), _off(l * _LBLKS)),
            out_shape=jax.ShapeDtypeStruct((_NROWS, _D), jnp.float32),
            input_output_aliases={6: 0},
        )(g, a, a, ws_t, wdc_t, wec_t, big)

    return big.reshape(_L, _B, _D)
